# Initial kernel scaffold; baseline (speedup 1.0000x reference)
import jax
import jax.numpy as jnp
from jax.experimental import pallas as pl


def kernel(inputs, table):
    raise NotImplementedError("write your pallas kernel here")


# speedup vs baseline: 1.4418x; 1.4418x over previous
import functools

import jax
import jax.numpy as jnp
from jax import lax
from jax.experimental import pallas as pl
from jax.experimental.pallas import tpu as pltpu
from jax.experimental.pallas import tpu_sc as plsc

B, S, F = 4096, 200, 27
N_CAT, E = 1000, 32
OUT_F = E + (F - 1)
L = 16
NC, NS = 2, 16
NW = NC * NS
N = B * S
PER_W = N // NW
C = 256
G = C // L
NCH = PER_W // C
HALF = NCH // 2
IN_CH = C * F
OUT_CH = C * OUT_F


def _build_sc_call():
    mesh = plsc.VectorSubcoreMesh(core_axis_name="c", subcore_axis_name="s")

    @functools.partial(
        pl.kernel,
        mesh=mesh,
        compiler_params=pltpu.CompilerParams(needs_layout_passes=False),
        out_type=jax.ShapeDtypeStruct((N * OUT_F,), jnp.float32),
        scratch_types=[
            pltpu.VMEM((N_CAT * E,), jnp.float32),
            pltpu.VMEM((IN_CH,), jnp.float32),
            pltpu.VMEM((IN_CH,), jnp.float32),
            pltpu.VMEM((OUT_CH,), jnp.float32),
            pltpu.VMEM((OUT_CH,), jnp.float32),
            pltpu.SemaphoreType.DMA,
            pltpu.SemaphoreType.DMA,
            pltpu.SemaphoreType.DMA,
            pltpu.SemaphoreType.DMA,
            pltpu.SemaphoreType.DMA,
        ],
    )
    def sc_fn(in_hbm, tab_hbm, out_hbm, tab_v, in0, in1, out0, out1,
              sem_t, sem_i0, sem_i1, sem_o0, sem_o1):
        wid = lax.axis_index("s") * NC + lax.axis_index("c")
        pos0 = wid * PER_W

        pltpu.async_copy(tab_hbm, tab_v, sem_t)
        pltpu.async_copy(in_hbm.at[pl.ds(pos0 * F, IN_CH)], in0, sem_i0)
        pltpu.async_copy(in_hbm.at[pl.ds((pos0 + C) * F, IN_CH)], in1, sem_i1)
        pltpu.make_async_copy(tab_hbm, tab_v, sem_t).wait()

        iota = lax.iota(jnp.int32, L)
        i27 = iota * F
        i58 = iota * OUT_F

        def compute(in_v, out_v):
            def gbody(g, carry):
                vin = i27 + g * (L * F)
                x = plsc.load_gather(in_v, [vin])
                y = x * 255.0
                t = y + 8388608.0
                rows = plsc.bitcast(t, jnp.int32) & 0x7FFFFF
                tb = rows * E
                ob = i58 + g * (L * OUT_F)
                for d in range(E):
                    v = plsc.load_gather(tab_v, [tb + d])
                    plsc.store_scatter(out_v, [ob + d], v)
                for j in range(1, F):
                    v = plsc.load_gather(in_v, [vin + j])
                    plsc.store_scatter(out_v, [ob + (E - 1 + j)], v)
                return carry
            lax.fori_loop(0, G, gbody, 0)

        def step(t_it, carry):
            for in_v, out_v, sem_i, sem_o, b in (
                    (in0, out0, sem_i0, sem_o0, 0),
                    (in1, out1, sem_i1, sem_o1, 1)):
                pos = pos0 + (2 * t_it + b) * C
                in_off = pos * F
                out_off = pos * OUT_F
                pltpu.make_async_copy(
                    in_hbm.at[pl.ds(in_off, IN_CH)], in_v, sem_i).wait()

                @pl.when(t_it > 0)
                def _wait_prev_store(out_v=out_v, out_off=out_off, sem_o=sem_o):
                    pltpu.make_async_copy(
                        out_v, out_hbm.at[pl.ds(out_off, OUT_CH)], sem_o).wait()

                compute(in_v, out_v)
                pltpu.async_copy(
                    out_v, out_hbm.at[pl.ds(out_off, OUT_CH)], sem_o)

                @pl.when(t_it < HALF - 1)
                def _start_next_load(in_v=in_v, in_off=in_off, sem_i=sem_i):
                    pltpu.async_copy(
                        in_hbm.at[pl.ds(in_off + 2 * IN_CH, IN_CH)], in_v, sem_i)
            return carry

        lax.fori_loop(0, HALF, step, 0)
        pltpu.make_async_copy(
            out0, out_hbm.at[pl.ds(pos0 * OUT_F, OUT_CH)], sem_o0).wait()
        pltpu.make_async_copy(
            out1, out_hbm.at[pl.ds(pos0 * OUT_F, OUT_CH)], sem_o1).wait()

    return sc_fn


_sc_call = _build_sc_call()


def kernel(inputs, table):
    out_flat = _sc_call(inputs.reshape(N * F), table.reshape(N_CAT * E))
    return out_flat.reshape(B, S, OUT_F)
